# Initial kernel scaffold; baseline (speedup 1.0000x reference)
import jax
import jax.numpy as jnp
from jax.experimental import pallas as pl


def kernel(x, edge_index, W1, b1, W2, b2, W3, b3):
    raise NotImplementedError("write your pallas kernel here")


# speedup vs baseline: 15.7227x; 15.7227x over previous
import functools

import jax
import jax.numpy as jnp
from jax import lax
from jax.experimental import pallas as pl
from jax.experimental.pallas import tpu as pltpu
from jax.experimental.pallas import tpu_sc as plsc

NCORE = 2
NSUB = 16
NW = NCORE * NSUB
CH = 128


@functools.lru_cache(None)
def _deg_kernel(np_, ep):
    ew = ep // NW
    nch = ew // CH
    rps = np_ // NSUB
    zr = 64
    mesh = plsc.VectorSubcoreMesh(core_axis_name="c", subcore_axis_name="s")

    def body(dst_hbm, out_hbm, dst_v, ones_v, zb_v, z_sh):
        c = lax.axis_index("c")
        s = lax.axis_index("s")
        one16 = jnp.ones((16,), jnp.float32)
        zero16 = jnp.zeros((16,), jnp.float32)
        for j in range(CH // 16):
            ones_v[pl.ds(j * 16, 16)] = one16
        for j in range(zr // 16):
            zb_v[pl.ds(j * 16, 16)] = zero16
        base = s * rps

        def zbody(i, carry):
            pltpu.sync_copy(zb_v, z_sh.at[pl.ds(base + i * zr, zr)])
            return carry

        lax.fori_loop(0, rps // zr, zbody, 0)
        plsc.subcore_barrier()
        ebase = (c * NSUB + s) * ew

        def ebody(k, carry):
            pltpu.sync_copy(dst_hbm.at[pl.ds(ebase + k * CH, CH)], dst_v)
            pltpu.sync_copy(ones_v, z_sh.at[dst_v], add=True)
            return carry

        lax.fori_loop(0, nch, ebody, 0)
        plsc.subcore_barrier()
        pltpu.sync_copy(z_sh.at[pl.ds(base, rps)],
                        out_hbm.at[pl.ds(c * np_ + base, rps)])

    return pl.kernel(
        body,
        out_type=jax.ShapeDtypeStruct((NCORE * np_,), jnp.float32),
        mesh=mesh,
        compiler_params=pltpu.CompilerParams(use_tc_tiling_on_sc=False),
        scratch_types=[
            pltpu.VMEM((CH,), jnp.int32),
            pltpu.VMEM((CH,), jnp.float32),
            pltpu.VMEM((zr,), jnp.float32),
            pltpu.VMEM_SHARED((np_,), jnp.float32),
        ],
    )


@functools.lru_cache(None)
def _agg_kernel(np_, ep, h):
    ew = ep // NW
    nch = ew // CH
    rps = np_ // NSUB
    zr = 32
    mesh = plsc.VectorSubcoreMesh(core_axis_name="c", subcore_axis_name="s")

    def body(y_hbm, src_hbm, dst_hbm, out_hbm, src_v, dst_v, rows_v, zb_v, z_sh):
        c = lax.axis_index("c")
        s = lax.axis_index("s")
        zero16 = jnp.zeros((16,), jnp.float32)
        for r in range(zr):
            for j in range(h // 16):
                zb_v[r, pl.ds(j * 16, 16)] = zero16
        base = s * rps

        def zbody(i, carry):
            pltpu.sync_copy(zb_v, z_sh.at[pl.ds(base + i * zr, zr)])
            return carry

        lax.fori_loop(0, rps // zr, zbody, 0)
        plsc.subcore_barrier()
        ebase = (c * NSUB + s) * ew

        def ebody(k, carry):
            off = ebase + k * CH
            pltpu.sync_copy(src_hbm.at[pl.ds(off, CH)], src_v)
            pltpu.sync_copy(dst_hbm.at[pl.ds(off, CH)], dst_v)
            pltpu.sync_copy(y_hbm.at[src_v], rows_v)
            pltpu.sync_copy(rows_v, z_sh.at[dst_v], add=True)
            return carry

        lax.fori_loop(0, nch, ebody, 0)
        plsc.subcore_barrier()
        pltpu.sync_copy(z_sh.at[pl.ds(base, rps)],
                        out_hbm.at[pl.ds(c * np_ + base, rps)])

    return pl.kernel(
        body,
        out_type=jax.ShapeDtypeStruct((NCORE * np_, h), jnp.float32),
        mesh=mesh,
        compiler_params=pltpu.CompilerParams(use_tc_tiling_on_sc=False),
        scratch_types=[
            pltpu.VMEM((CH,), jnp.int32),
            pltpu.VMEM((CH,), jnp.int32),
            pltpu.VMEM((CH, h), jnp.float32),
            pltpu.VMEM((zr, h), jnp.float32),
            pltpu.VMEM_SHARED((np_, h), jnp.float32),
        ],
    )


@functools.lru_cache(None)
def _dinv_kernel(np_):
    def body(degp_ref, out_ref):
        deg = degp_ref[0, :] + degp_ref[1, :] + 1.0
        out_ref[...] = lax.rsqrt(deg)

    return pl.pallas_call(
        body, out_shape=jax.ShapeDtypeStruct((np_,), jnp.float32))


@functools.lru_cache(None)
def _first_tc(np_, f_in, h1):
    r = 1024

    def body(x_ref, w_ref, dinv_ref, out_ref):
        xw = jnp.dot(x_ref[...], w_ref[...],
                     preferred_element_type=jnp.float32,
                     precision=lax.Precision.HIGHEST)
        out_ref[...] = xw * dinv_ref[...]

    return pl.pallas_call(
        body,
        grid=(np_ // r,),
        in_specs=[
            pl.BlockSpec((r, f_in), lambda i: (i, 0)),
            pl.BlockSpec((f_in, h1), lambda i: (0, 0)),
            pl.BlockSpec((r, 1), lambda i: (i, 0)),
        ],
        out_specs=pl.BlockSpec((r, h1), lambda i: (i, 0)),
        out_shape=jax.ShapeDtypeStruct((np_, h1), jnp.float32),
    )


@functools.lru_cache(None)
def _mid_tc(np_, hin, hout):
    r = 1024

    def body(z0_ref, z1_ref, y_ref, b_ref, w_ref, dinv_ref, out_ref):
        dinv = dinv_ref[...]
        agg = z0_ref[...] + z1_ref[...] + y_ref[...]
        hcur = jnp.maximum(agg * dinv + b_ref[...], 0.0)
        out_ref[...] = jnp.dot(hcur, w_ref[...],
                               preferred_element_type=jnp.float32,
                               precision=lax.Precision.HIGHEST) * dinv

    return pl.pallas_call(
        body,
        grid=(np_ // r,),
        in_specs=[
            pl.BlockSpec((r, hin), lambda i: (i, 0)),
            pl.BlockSpec((r, hin), lambda i: (i, 0)),
            pl.BlockSpec((r, hin), lambda i: (i, 0)),
            pl.BlockSpec((1, hin), lambda i: (0, 0)),
            pl.BlockSpec((hin, hout), lambda i: (0, 0)),
            pl.BlockSpec((r, 1), lambda i: (i, 0)),
        ],
        out_specs=pl.BlockSpec((r, hout), lambda i: (i, 0)),
        out_shape=jax.ShapeDtypeStruct((np_, hout), jnp.float32),
    )


@functools.lru_cache(None)
def _final_tc(n, hp, c_out):
    r = 1000

    def body(z0_ref, z1_ref, y_ref, b_ref, dinv_ref, out_ref):
        agg = z0_ref[...] + z1_ref[...] + y_ref[...]
        res = agg * dinv_ref[...] + b_ref[...]
        out_ref[...] = res[:, :c_out]

    return pl.pallas_call(
        body,
        grid=(n // r,),
        in_specs=[
            pl.BlockSpec((r, hp), lambda i: (i, 0)),
            pl.BlockSpec((r, hp), lambda i: (i, 0)),
            pl.BlockSpec((r, hp), lambda i: (i, 0)),
            pl.BlockSpec((1, hp), lambda i: (0, 0)),
            pl.BlockSpec((r, 1), lambda i: (i, 0)),
        ],
        out_specs=pl.BlockSpec((r, c_out), lambda i: (i, 0)),
        out_shape=jax.ShapeDtypeStruct((n, c_out), jnp.float32),
    )


def kernel(x, edge_index, W1, b1, W2, b2, W3, b3):
    n, f_in = x.shape
    e = edge_index.shape[1]
    h1, h2, c_out = W1.shape[1], W2.shape[1], W3.shape[1]
    hp = -(-c_out // 16) * 16

    np_ = (n // 512 + 1) * 512
    ep = -(-e // (NW * CH)) * (NW * CH)

    src = edge_index[0]
    dst = edge_index[1]
    pad = ep - e
    if pad:
        fill = jnp.full((pad,), n, dtype=src.dtype)
        src = jnp.concatenate([src, fill])
        dst = jnp.concatenate([dst, fill])
    xp = jnp.pad(x, ((0, np_ - n), (0, 0)))
    w3p = jnp.pad(W3, ((0, 0), (0, hp - c_out)))
    b3p = jnp.pad(b3, (0, hp - c_out)).reshape(1, hp)

    degp = _deg_kernel(np_, ep)(dst)
    dinv2 = _dinv_kernel(np_)(degp.reshape(NCORE, np_)).reshape(np_, 1)

    y1 = _first_tc(np_, f_in, h1)(xp, W1, dinv2)
    z1 = _agg_kernel(np_, ep, h1)(y1, src, dst)
    y2 = _mid_tc(np_, h1, h2)(z1[:np_], z1[np_:], y1, b1.reshape(1, h1), W2, dinv2)
    z2 = _agg_kernel(np_, ep, h2)(y2, src, dst)
    y3 = _mid_tc(np_, h2, hp)(z2[:np_], z2[np_:], y2, b2.reshape(1, h2), w3p, dinv2)
    z3 = _agg_kernel(np_, ep, hp)(y3, src, dst)
    return _final_tc(n, hp, c_out)(z3[:np_], z3[np_:], y3, b3p, dinv2)


# speedup vs baseline: 19.0395x; 1.2110x over previous
import functools

import jax
import jax.numpy as jnp
from jax import lax
from jax.experimental import pallas as pl
from jax.experimental.pallas import tpu as pltpu
from jax.experimental.pallas import tpu_sc as plsc

NCORE = 2
NSUB = 16
NW = NCORE * NSUB
CH = 128


@functools.lru_cache(None)
def _deg_kernel(np_, ep):
    ew = ep // NW
    nch = ew // CH
    rps = np_ // NSUB
    zr = 64
    mesh = plsc.VectorSubcoreMesh(core_axis_name="c", subcore_axis_name="s")

    def body(dst_hbm, out_hbm, dst_v, ones_v, zb_v, z_sh):
        c = lax.axis_index("c")
        s = lax.axis_index("s")
        one16 = jnp.ones((16,), jnp.float32)
        zero16 = jnp.zeros((16,), jnp.float32)
        for j in range(CH // 16):
            ones_v[pl.ds(j * 16, 16)] = one16
        for j in range(zr // 16):
            zb_v[pl.ds(j * 16, 16)] = zero16
        base = s * rps

        def zbody(i, carry):
            pltpu.sync_copy(zb_v, z_sh.at[pl.ds(base + i * zr, zr)])
            return carry

        lax.fori_loop(0, rps // zr, zbody, 0)
        plsc.subcore_barrier()
        ebase = (c * NSUB + s) * ew

        def ebody(k, carry):
            pltpu.sync_copy(dst_hbm.at[pl.ds(ebase + k * CH, CH)], dst_v)
            pltpu.sync_copy(ones_v, z_sh.at[dst_v], add=True)
            return carry

        lax.fori_loop(0, nch, ebody, 0)
        plsc.subcore_barrier()
        pltpu.sync_copy(z_sh.at[pl.ds(base, rps)],
                        out_hbm.at[pl.ds(c * np_ + base, rps)])

    return pl.kernel(
        body,
        out_type=jax.ShapeDtypeStruct((NCORE * np_,), jnp.float32),
        mesh=mesh,
        compiler_params=pltpu.CompilerParams(use_tc_tiling_on_sc=False),
        scratch_types=[
            pltpu.VMEM((CH,), jnp.int32),
            pltpu.VMEM((CH,), jnp.float32),
            pltpu.VMEM((zr,), jnp.float32),
            pltpu.VMEM_SHARED((np_,), jnp.float32),
        ],
    )


@functools.lru_cache(None)
def _agg_kernel(np_, ep, h):
    ew = ep // NW
    nch = ew // CH
    assert nch % 2 == 0
    rps = np_ // NSUB
    zr = 128
    assert rps % zr == 0
    mesh = plsc.VectorSubcoreMesh(core_axis_name="c", subcore_axis_name="s")

    def body(y_hbm, src_hbm, dst_hbm, out_hbm,
             src0, src1, dst0, dst1, rows0, rows1, zb_v, z_sh,
             sg0, sg1, ss0, ss1, sd0, sd1):
        c = lax.axis_index("c")
        s = lax.axis_index("s")
        src_b = (src0, src1)
        dst_b = (dst0, dst1)
        rows_b = (rows0, rows1)
        sg = (sg0, sg1)
        ss = (ss0, ss1)
        sd = (sd0, sd1)
        zero16 = jnp.zeros((16,), jnp.float32)
        for r in range(zr):
            for j in range(h // 16):
                zb_v[r, pl.ds(j * 16, 16)] = zero16
        base = s * rps

        def zbody(i, carry):
            pltpu.sync_copy(zb_v, z_sh.at[pl.ds(base + i * zr, zr)])
            return carry

        lax.fori_loop(0, rps // zr, zbody, 0)
        plsc.subcore_barrier()
        ebase = (c * NSUB + s) * ew
        last = ebase + (nch - 1) * CH

        pltpu.sync_copy(src_hbm.at[pl.ds(ebase, CH)], src0)
        pltpu.sync_copy(dst_hbm.at[pl.ds(ebase, CH)], dst0)
        pltpu.async_copy(y_hbm.at[src0], rows0, sg0)
        pltpu.async_copy(src_hbm.at[pl.ds(ebase + CH, CH)], src1, ss1)
        pltpu.async_copy(dst_hbm.at[pl.ds(ebase + CH, CH)], dst1, sd1)

        def step(k, p):
            q = 1 - p
            off_g = jnp.minimum(ebase + (k + 1) * CH, last)
            off_i = jnp.minimum(ebase + (k + 2) * CH, last)
            pltpu.make_async_copy(y_hbm.at[src_b[p]], rows_b[p], sg[p]).wait()
            pltpu.make_async_copy(src_hbm.at[pl.ds(off_g, CH)], src_b[q], ss[q]).wait()
            pltpu.make_async_copy(dst_hbm.at[pl.ds(off_g, CH)], dst_b[q], sd[q]).wait()
            pltpu.async_copy(y_hbm.at[src_b[q]], rows_b[q], sg[q])
            pltpu.sync_copy(rows_b[p], z_sh.at[dst_b[p]], add=True)
            pltpu.async_copy(src_hbm.at[pl.ds(off_i, CH)], src_b[p], ss[p])
            pltpu.async_copy(dst_hbm.at[pl.ds(off_i, CH)], dst_b[p], sd[p])

        def ebody(g, carry):
            step(2 * g, 0)
            step(2 * g + 1, 1)
            return carry

        lax.fori_loop(0, nch // 2, ebody, 0)
        pltpu.make_async_copy(y_hbm.at[src_b[0]], rows_b[0], sg[0]).wait()
        pltpu.make_async_copy(src_hbm.at[pl.ds(last, CH)], src_b[1], ss[1]).wait()
        pltpu.make_async_copy(dst_hbm.at[pl.ds(last, CH)], dst_b[1], sd[1]).wait()
        plsc.subcore_barrier()
        pltpu.sync_copy(z_sh.at[pl.ds(base, rps)],
                        out_hbm.at[pl.ds(c * np_ + base, rps)])

    return pl.kernel(
        body,
        out_type=jax.ShapeDtypeStruct((NCORE * np_, h), jnp.float32),
        mesh=mesh,
        compiler_params=pltpu.CompilerParams(use_tc_tiling_on_sc=False),
        scratch_types=[
            pltpu.VMEM((CH,), jnp.int32),
            pltpu.VMEM((CH,), jnp.int32),
            pltpu.VMEM((CH,), jnp.int32),
            pltpu.VMEM((CH,), jnp.int32),
            pltpu.VMEM((CH, h), jnp.float32),
            pltpu.VMEM((CH, h), jnp.float32),
            pltpu.VMEM((zr, h), jnp.float32),
            pltpu.VMEM_SHARED((np_, h), jnp.float32),
            pltpu.SemaphoreType.DMA,
            pltpu.SemaphoreType.DMA,
            pltpu.SemaphoreType.DMA,
            pltpu.SemaphoreType.DMA,
            pltpu.SemaphoreType.DMA,
            pltpu.SemaphoreType.DMA,
        ],
    )


@functools.lru_cache(None)
def _dinv_kernel(np_):
    def body(degp_ref, out_ref):
        deg = degp_ref[0, :] + degp_ref[1, :] + 1.0
        out_ref[...] = lax.rsqrt(deg)

    return pl.pallas_call(
        body, out_shape=jax.ShapeDtypeStruct((np_,), jnp.float32))


@functools.lru_cache(None)
def _first_tc(np_, f_in, h1):
    r = 1024

    def body(x_ref, w_ref, dinv_ref, out_ref):
        xw = jnp.dot(x_ref[...], w_ref[...],
                     preferred_element_type=jnp.float32,
                     precision=lax.Precision.HIGHEST)
        out_ref[...] = xw * dinv_ref[...]

    return pl.pallas_call(
        body,
        grid=(np_ // r,),
        in_specs=[
            pl.BlockSpec((r, f_in), lambda i: (i, 0)),
            pl.BlockSpec((f_in, h1), lambda i: (0, 0)),
            pl.BlockSpec((r, 1), lambda i: (i, 0)),
        ],
        out_specs=pl.BlockSpec((r, h1), lambda i: (i, 0)),
        out_shape=jax.ShapeDtypeStruct((np_, h1), jnp.float32),
    )


@functools.lru_cache(None)
def _mid_tc(np_, hin, hout):
    r = 1024

    def body(z0_ref, z1_ref, y_ref, b_ref, w_ref, dinv_ref, out_ref):
        dinv = dinv_ref[...]
        agg = z0_ref[...] + z1_ref[...] + y_ref[...]
        hcur = jnp.maximum(agg * dinv + b_ref[...], 0.0)
        out_ref[...] = jnp.dot(hcur, w_ref[...],
                               preferred_element_type=jnp.float32,
                               precision=lax.Precision.HIGHEST) * dinv

    return pl.pallas_call(
        body,
        grid=(np_ // r,),
        in_specs=[
            pl.BlockSpec((r, hin), lambda i: (i, 0)),
            pl.BlockSpec((r, hin), lambda i: (i, 0)),
            pl.BlockSpec((r, hin), lambda i: (i, 0)),
            pl.BlockSpec((1, hin), lambda i: (0, 0)),
            pl.BlockSpec((hin, hout), lambda i: (0, 0)),
            pl.BlockSpec((r, 1), lambda i: (i, 0)),
        ],
        out_specs=pl.BlockSpec((r, hout), lambda i: (i, 0)),
        out_shape=jax.ShapeDtypeStruct((np_, hout), jnp.float32),
    )


@functools.lru_cache(None)
def _final_tc(n, hp, c_out):
    r = 1000

    def body(z0_ref, z1_ref, y_ref, b_ref, dinv_ref, out_ref):
        agg = z0_ref[...] + z1_ref[...] + y_ref[...]
        res = agg * dinv_ref[...] + b_ref[...]
        out_ref[...] = res[:, :c_out]

    return pl.pallas_call(
        body,
        grid=(n // r,),
        in_specs=[
            pl.BlockSpec((r, hp), lambda i: (i, 0)),
            pl.BlockSpec((r, hp), lambda i: (i, 0)),
            pl.BlockSpec((r, hp), lambda i: (i, 0)),
            pl.BlockSpec((1, hp), lambda i: (0, 0)),
            pl.BlockSpec((r, 1), lambda i: (i, 0)),
        ],
        out_specs=pl.BlockSpec((r, c_out), lambda i: (i, 0)),
        out_shape=jax.ShapeDtypeStruct((n, c_out), jnp.float32),
    )


def kernel(x, edge_index, W1, b1, W2, b2, W3, b3):
    n, f_in = x.shape
    e = edge_index.shape[1]
    h1, h2, c_out = W1.shape[1], W2.shape[1], W3.shape[1]
    hp = -(-c_out // 16) * 16

    np_ = (n // 512 + 1) * 512
    ep = -(-e // (NW * CH * 2)) * (NW * CH * 2)

    src = edge_index[0]
    dst = edge_index[1]
    pad = ep - e
    if pad:
        fill = jnp.full((pad,), n, dtype=src.dtype)
        src = jnp.concatenate([src, fill])
        dst = jnp.concatenate([dst, fill])
    xp = jnp.pad(x, ((0, np_ - n), (0, 0)))
    w3p = jnp.pad(W3, ((0, 0), (0, hp - c_out)))
    b3p = jnp.pad(b3, (0, hp - c_out)).reshape(1, hp)

    degp = _deg_kernel(np_, ep)(dst)
    dinv2 = _dinv_kernel(np_)(degp.reshape(NCORE, np_)).reshape(np_, 1)

    y1 = _first_tc(np_, f_in, h1)(xp, W1, dinv2)
    z1 = _agg_kernel(np_, ep, h1)(y1, src, dst)
    y2 = _mid_tc(np_, h1, h2)(z1[:np_], z1[np_:], y1, b1.reshape(1, h1), W2, dinv2)
    z2 = _agg_kernel(np_, ep, h2)(y2, src, dst)
    y3 = _mid_tc(np_, h2, hp)(z2[:np_], z2[np_:], y2, b2.reshape(1, h2), w3p, dinv2)
    z3 = _agg_kernel(np_, ep, hp)(y3, src, dst)
    return _final_tc(n, hp, c_out)(z3[:np_], z3[np_:], y3, b3p, dinv2)


# speedup vs baseline: 20.2635x; 1.0643x over previous
import functools

import jax
import jax.numpy as jnp
from jax import lax
from jax.experimental import pallas as pl
from jax.experimental.pallas import tpu as pltpu
from jax.experimental.pallas import tpu_sc as plsc

NCORE = 2
NSUB = 16
NW = NCORE * NSUB
CH = 128


@functools.lru_cache(None)
def _deg_kernel(np_, ep):
    ew = ep // NW
    nch = ew // CH
    rps = np_ // NSUB
    zr = 64
    mesh = plsc.VectorSubcoreMesh(core_axis_name="c", subcore_axis_name="s")

    def body(dst_hbm, out_hbm, dst_v, ones_v, zb_v, z_sh):
        c = lax.axis_index("c")
        s = lax.axis_index("s")
        one16 = jnp.ones((16,), jnp.float32)
        zero16 = jnp.zeros((16,), jnp.float32)
        for j in range(CH // 16):
            ones_v[pl.ds(j * 16, 16)] = one16
        for j in range(zr // 16):
            zb_v[pl.ds(j * 16, 16)] = zero16
        base = s * rps

        def zbody(i, carry):
            pltpu.sync_copy(zb_v, z_sh.at[pl.ds(base + i * zr, zr)])
            return carry

        lax.fori_loop(0, rps // zr, zbody, 0)
        plsc.subcore_barrier()
        ebase = (c * NSUB + s) * ew

        def ebody(k, carry):
            pltpu.sync_copy(dst_hbm.at[pl.ds(ebase + k * CH, CH)], dst_v)
            pltpu.sync_copy(ones_v, z_sh.at[dst_v], add=True)
            return carry

        lax.fori_loop(0, nch, ebody, 0)
        plsc.subcore_barrier()
        pltpu.sync_copy(z_sh.at[pl.ds(base, rps)],
                        out_hbm.at[pl.ds(c * np_ + base, rps)])

    return pl.kernel(
        body,
        out_type=jax.ShapeDtypeStruct((NCORE * np_,), jnp.float32),
        mesh=mesh,
        compiler_params=pltpu.CompilerParams(use_tc_tiling_on_sc=False),
        scratch_types=[
            pltpu.VMEM((CH,), jnp.int32),
            pltpu.VMEM((CH,), jnp.float32),
            pltpu.VMEM((zr,), jnp.float32),
            pltpu.VMEM_SHARED((np_,), jnp.float32),
        ],
    )


@functools.lru_cache(None)
def _agg_kernel(np_, ep, h):
    ew = ep // NW
    nch = ew // CH
    assert nch % 4 == 0
    ng = nch // 2
    rps = np_ // NSUB
    zr = 128
    assert rps % zr == 0
    mesh = plsc.VectorSubcoreMesh(core_axis_name="c", subcore_axis_name="s")

    def body(y_hbm, src_hbm, dst_hbm, out_hbm,
             src0, src1, src2, src3, dst0, dst1, dst2, dst3,
             rows0, rows1, rows2, rows3, zb_v, z_sh,
             sg0, sg1, sg2, sg3, ss0, ss1, ss2, ss3,
             sd0, sd1, sd2, sd3, sc0, sc1):
        c = lax.axis_index("c")
        s = lax.axis_index("s")
        src_b = (src0, src1, src2, src3)
        dst_b = (dst0, dst1, dst2, dst3)
        rows_b = (rows0, rows1, rows2, rows3)
        sg = (sg0, sg1, sg2, sg3)
        ss = (ss0, ss1, ss2, ss3)
        sd = (sd0, sd1, sd2, sd3)
        zero16 = jnp.zeros((16,), jnp.float32)
        for r in range(zr):
            for j in range(h // 16):
                zb_v[r, pl.ds(j * 16, 16)] = zero16
        base = s * rps

        def zbody(i, carry):
            pltpu.sync_copy(zb_v, z_sh.at[pl.ds(base + i * zr, zr)])
            return carry

        lax.fori_loop(0, rps // zr, zbody, 0)
        plsc.subcore_barrier()
        ebase = (c * NSUB + s) * ew
        last = ebase + (nch - 1) * CH

        def idx_off(m):
            return jnp.minimum(ebase + m * CH, last)

        pltpu.sync_copy(src_hbm.at[pl.ds(ebase, CH)], src0)
        pltpu.sync_copy(dst_hbm.at[pl.ds(ebase, CH)], dst0)
        pltpu.sync_copy(src_hbm.at[pl.ds(ebase + CH, CH)], src1)
        pltpu.sync_copy(dst_hbm.at[pl.ds(ebase + CH, CH)], dst1)
        pltpu.async_copy(y_hbm.at[src0], rows0, sg0)
        pltpu.async_copy(y_hbm.at[src1], rows1, sg1)
        pltpu.async_copy(src_hbm.at[pl.ds(ebase + 2 * CH, CH)], src2, ss2)
        pltpu.async_copy(dst_hbm.at[pl.ds(ebase + 2 * CH, CH)], dst2, sd2)
        pltpu.async_copy(src_hbm.at[pl.ds(ebase + 3 * CH, CH)], src3, ss3)
        pltpu.async_copy(dst_hbm.at[pl.ds(ebase + 3 * CH, CH)], dst3, sd3)

        def step(g, p):
            q = 1 - p
            a, b = 2 * p, 2 * p + 1
            e0, e1 = 2 * q, 2 * q + 1
            pltpu.make_async_copy(y_hbm.at[src_b[a]], rows_b[a], sg[a]).wait()
            pltpu.make_async_copy(y_hbm.at[src_b[b]], rows_b[b], sg[b]).wait()
            o2, o3 = idx_off(2 * g + 2), idx_off(2 * g + 3)
            pltpu.make_async_copy(src_hbm.at[pl.ds(o2, CH)], src_b[e0], ss[e0]).wait()
            pltpu.make_async_copy(dst_hbm.at[pl.ds(o2, CH)], dst_b[e0], sd[e0]).wait()
            pltpu.make_async_copy(src_hbm.at[pl.ds(o3, CH)], src_b[e1], ss[e1]).wait()
            pltpu.make_async_copy(dst_hbm.at[pl.ds(o3, CH)], dst_b[e1], sd[e1]).wait()
            pltpu.async_copy(y_hbm.at[src_b[e0]], rows_b[e0], sg[e0])
            pltpu.async_copy(y_hbm.at[src_b[e1]], rows_b[e1], sg[e1])
            pltpu.async_copy(rows_b[a], z_sh.at[dst_b[a]], sc0, add=True)
            pltpu.async_copy(rows_b[b], z_sh.at[dst_b[b]], sc1, add=True)
            pltpu.make_async_copy(rows_b[a], z_sh.at[dst_b[a]], sc0).wait()
            pltpu.make_async_copy(rows_b[b], z_sh.at[dst_b[b]], sc1).wait()
            o4, o5 = idx_off(2 * g + 4), idx_off(2 * g + 5)
            pltpu.async_copy(src_hbm.at[pl.ds(o4, CH)], src_b[a], ss[a])
            pltpu.async_copy(dst_hbm.at[pl.ds(o4, CH)], dst_b[a], sd[a])
            pltpu.async_copy(src_hbm.at[pl.ds(o5, CH)], src_b[b], ss[b])
            pltpu.async_copy(dst_hbm.at[pl.ds(o5, CH)], dst_b[b], sd[b])

        def ebody(gg, carry):
            step(2 * gg, 0)
            step(2 * gg + 1, 1)
            return carry

        lax.fori_loop(0, ng // 2, ebody, 0)
        p_last = (ng - 1) & 1
        q_last = 1 - p_last
        for i in (2 * q_last, 2 * q_last + 1):
            pltpu.make_async_copy(y_hbm.at[src_b[i]], rows_b[i], sg[i]).wait()
        for i in (2 * p_last, 2 * p_last + 1):
            pltpu.make_async_copy(src_hbm.at[pl.ds(last, CH)], src_b[i], ss[i]).wait()
            pltpu.make_async_copy(dst_hbm.at[pl.ds(last, CH)], dst_b[i], sd[i]).wait()
        plsc.subcore_barrier()
        pltpu.sync_copy(z_sh.at[pl.ds(base, rps)],
                        out_hbm.at[pl.ds(c * np_ + base, rps)])

    return pl.kernel(
        body,
        out_type=jax.ShapeDtypeStruct((NCORE * np_, h), jnp.float32),
        mesh=mesh,
        compiler_params=pltpu.CompilerParams(use_tc_tiling_on_sc=False),
        scratch_types=(
            [pltpu.VMEM((CH,), jnp.int32)] * 8
            + [pltpu.VMEM((CH, h), jnp.float32)] * 4
            + [pltpu.VMEM((zr, h), jnp.float32),
               pltpu.VMEM_SHARED((np_, h), jnp.float32)]
            + [pltpu.SemaphoreType.DMA] * 14
        ),
    )


@functools.lru_cache(None)
def _dinv_kernel(np_):
    def body(degp_ref, out_ref):
        deg = degp_ref[0, :] + degp_ref[1, :] + 1.0
        out_ref[...] = lax.rsqrt(deg)

    return pl.pallas_call(
        body, out_shape=jax.ShapeDtypeStruct((np_,), jnp.float32))


@functools.lru_cache(None)
def _first_tc(np_, f_in, h1):
    r = 1024

    def body(x_ref, w_ref, dinv_ref, out_ref):
        xw = jnp.dot(x_ref[...], w_ref[...],
                     preferred_element_type=jnp.float32,
                     precision=lax.Precision.HIGHEST)
        out_ref[...] = xw * dinv_ref[...]

    return pl.pallas_call(
        body,
        grid=(np_ // r,),
        in_specs=[
            pl.BlockSpec((r, f_in), lambda i: (i, 0)),
            pl.BlockSpec((f_in, h1), lambda i: (0, 0)),
            pl.BlockSpec((r, 1), lambda i: (i, 0)),
        ],
        out_specs=pl.BlockSpec((r, h1), lambda i: (i, 0)),
        out_shape=jax.ShapeDtypeStruct((np_, h1), jnp.float32),
    )


@functools.lru_cache(None)
def _mid_tc(np_, hin, hout):
    r = 1024

    def body(z0_ref, z1_ref, y_ref, b_ref, w_ref, dinv_ref, out_ref):
        dinv = dinv_ref[...]
        agg = z0_ref[...] + z1_ref[...] + y_ref[...]
        hcur = jnp.maximum(agg * dinv + b_ref[...], 0.0)
        out_ref[...] = jnp.dot(hcur, w_ref[...],
                               preferred_element_type=jnp.float32,
                               precision=lax.Precision.HIGHEST) * dinv

    return pl.pallas_call(
        body,
        grid=(np_ // r,),
        in_specs=[
            pl.BlockSpec((r, hin), lambda i: (i, 0)),
            pl.BlockSpec((r, hin), lambda i: (i, 0)),
            pl.BlockSpec((r, hin), lambda i: (i, 0)),
            pl.BlockSpec((1, hin), lambda i: (0, 0)),
            pl.BlockSpec((hin, hout), lambda i: (0, 0)),
            pl.BlockSpec((r, 1), lambda i: (i, 0)),
        ],
        out_specs=pl.BlockSpec((r, hout), lambda i: (i, 0)),
        out_shape=jax.ShapeDtypeStruct((np_, hout), jnp.float32),
    )


@functools.lru_cache(None)
def _final_tc(n, hp, c_out):
    r = 1000

    def body(z0_ref, z1_ref, y_ref, b_ref, dinv_ref, out_ref):
        agg = z0_ref[...] + z1_ref[...] + y_ref[...]
        res = agg * dinv_ref[...] + b_ref[...]
        out_ref[...] = res[:, :c_out]

    return pl.pallas_call(
        body,
        grid=(n // r,),
        in_specs=[
            pl.BlockSpec((r, hp), lambda i: (i, 0)),
            pl.BlockSpec((r, hp), lambda i: (i, 0)),
            pl.BlockSpec((r, hp), lambda i: (i, 0)),
            pl.BlockSpec((1, hp), lambda i: (0, 0)),
            pl.BlockSpec((r, 1), lambda i: (i, 0)),
        ],
        out_specs=pl.BlockSpec((r, c_out), lambda i: (i, 0)),
        out_shape=jax.ShapeDtypeStruct((n, c_out), jnp.float32),
    )


def kernel(x, edge_index, W1, b1, W2, b2, W3, b3):
    n, f_in = x.shape
    e = edge_index.shape[1]
    h1, h2, c_out = W1.shape[1], W2.shape[1], W3.shape[1]
    hp = -(-c_out // 16) * 16

    np_ = (n // 512 + 1) * 512
    ep = -(-e // (NW * CH * 4)) * (NW * CH * 4)

    src = edge_index[0]
    dst = edge_index[1]
    pad = ep - e
    if pad:
        fill = jnp.full((pad,), n, dtype=src.dtype)
        src = jnp.concatenate([src, fill])
        dst = jnp.concatenate([dst, fill])
    xp = jnp.pad(x, ((0, np_ - n), (0, 0)))
    w3p = jnp.pad(W3, ((0, 0), (0, hp - c_out)))
    b3p = jnp.pad(b3, (0, hp - c_out)).reshape(1, hp)

    degp = _deg_kernel(np_, ep)(dst)
    dinv2 = _dinv_kernel(np_)(degp.reshape(NCORE, np_)).reshape(np_, 1)

    y1 = _first_tc(np_, f_in, h1)(xp, W1, dinv2)
    z1 = _agg_kernel(np_, ep, h1)(y1, src, dst)
    y2 = _mid_tc(np_, h1, h2)(z1[:np_], z1[np_:], y1, b1.reshape(1, h1), W2, dinv2)
    z2 = _agg_kernel(np_, ep, h2)(y2, src, dst)
    y3 = _mid_tc(np_, h2, hp)(z2[:np_], z2[np_:], y2, b2.reshape(1, h2), w3p, dinv2)
    z3 = _agg_kernel(np_, ep, hp)(y3, src, dst)
    return _final_tc(n, hp, c_out)(z3[:np_], z3[np_:], y3, b3p, dinv2)


# speedup vs baseline: 30.7336x; 1.5167x over previous
import functools

import jax
import jax.numpy as jnp
from jax import lax
from jax.experimental import pallas as pl
from jax.experimental.pallas import tpu as pltpu
from jax.experimental.pallas import tpu_sc as plsc

NCORE = 2
NSUB = 16
NW = NCORE * NSUB
CH = 128


@functools.lru_cache(None)
def _deg_kernel(np_, ep):
    ew = ep // NW
    nch = ew // CH
    rps = np_ // NSUB
    zr = 64
    mesh = plsc.VectorSubcoreMesh(core_axis_name="c", subcore_axis_name="s")

    def body(dst_hbm, out_hbm, dst_v, ones_v, zb_v, z_sh):
        c = lax.axis_index("c")
        s = lax.axis_index("s")
        one16 = jnp.ones((16,), jnp.float32)
        zero16 = jnp.zeros((16,), jnp.float32)
        for j in range(CH // 16):
            ones_v[pl.ds(j * 16, 16)] = one16
        for j in range(zr // 16):
            zb_v[pl.ds(j * 16, 16)] = zero16
        base = s * rps

        def zbody(i, carry):
            pltpu.sync_copy(zb_v, z_sh.at[pl.ds(base + i * zr, zr)])
            return carry

        lax.fori_loop(0, rps // zr, zbody, 0)
        plsc.subcore_barrier()
        ebase = (c * NSUB + s) * ew

        def ebody(k, carry):
            pltpu.sync_copy(dst_hbm.at[pl.ds(ebase + k * CH, CH)], dst_v)
            pltpu.sync_copy(ones_v, z_sh.at[dst_v], add=True)
            return carry

        lax.fori_loop(0, nch, ebody, 0)
        plsc.subcore_barrier()
        pltpu.sync_copy(z_sh.at[pl.ds(base, rps)],
                        out_hbm.at[pl.ds(c * np_ + base, rps)])

    return pl.kernel(
        body,
        out_type=jax.ShapeDtypeStruct((NCORE * np_,), jnp.float32),
        mesh=mesh,
        compiler_params=pltpu.CompilerParams(use_tc_tiling_on_sc=False),
        scratch_types=[
            pltpu.VMEM((CH,), jnp.int32),
            pltpu.VMEM((CH,), jnp.float32),
            pltpu.VMEM((zr,), jnp.float32),
            pltpu.VMEM_SHARED((np_,), jnp.float32),
        ],
    )


@functools.lru_cache(None)
def _agg_kernel(np_, ep, h):
    ew = ep // NW
    nch = ew // CH
    assert nch % 4 == 0
    ng = nch // 2
    rps = np_ // NSUB
    zr = 128
    assert rps % zr == 0
    mesh = plsc.VectorSubcoreMesh(core_axis_name="c", subcore_axis_name="s")

    def body(y_hbm, src_hbm, dst_hbm, out_hbm,
             src0, src1, src2, src3, dst0, dst1, dst2, dst3,
             rows0, rows1, rows2, rows3, zb_v, z_sh, y_sh,
             sg0, sg1, sg2, sg3, ss0, ss1, ss2, ss3,
             sd0, sd1, sd2, sd3, sc0, sc1):
        c = lax.axis_index("c")
        s = lax.axis_index("s")
        src_b = (src0, src1, src2, src3)
        dst_b = (dst0, dst1, dst2, dst3)
        rows_b = (rows0, rows1, rows2, rows3)
        sg = (sg0, sg1, sg2, sg3)
        ss = (ss0, ss1, ss2, ss3)
        sd = (sd0, sd1, sd2, sd3)
        zero16 = jnp.zeros((16,), jnp.float32)
        for r in range(zr):
            for j in range(h // 16):
                zb_v[r, pl.ds(j * 16, 16)] = zero16
        base = s * rps

        def zbody(i, carry):
            pltpu.sync_copy(zb_v, z_sh.at[pl.ds(base + i * zr, zr)])
            return carry

        lax.fori_loop(0, rps // zr, zbody, 0)
        pltpu.sync_copy(y_hbm.at[pl.ds(base, rps)], y_sh.at[pl.ds(base, rps)])
        plsc.subcore_barrier()
        ebase = (c * NSUB + s) * ew
        last = ebase + (nch - 1) * CH

        def idx_off(m):
            return jnp.minimum(ebase + m * CH, last)

        pltpu.sync_copy(src_hbm.at[pl.ds(ebase, CH)], src0)
        pltpu.sync_copy(dst_hbm.at[pl.ds(ebase, CH)], dst0)
        pltpu.sync_copy(src_hbm.at[pl.ds(ebase + CH, CH)], src1)
        pltpu.sync_copy(dst_hbm.at[pl.ds(ebase + CH, CH)], dst1)
        pltpu.async_copy(y_sh.at[src0], rows0, sg0)
        pltpu.async_copy(y_sh.at[src1], rows1, sg1)
        pltpu.async_copy(src_hbm.at[pl.ds(ebase + 2 * CH, CH)], src2, ss2)
        pltpu.async_copy(dst_hbm.at[pl.ds(ebase + 2 * CH, CH)], dst2, sd2)
        pltpu.async_copy(src_hbm.at[pl.ds(ebase + 3 * CH, CH)], src3, ss3)
        pltpu.async_copy(dst_hbm.at[pl.ds(ebase + 3 * CH, CH)], dst3, sd3)

        def step(g, p):
            q = 1 - p
            a, b = 2 * p, 2 * p + 1
            e0, e1 = 2 * q, 2 * q + 1
            pltpu.make_async_copy(y_sh.at[src_b[a]], rows_b[a], sg[a]).wait()
            pltpu.make_async_copy(y_sh.at[src_b[b]], rows_b[b], sg[b]).wait()
            o2, o3 = idx_off(2 * g + 2), idx_off(2 * g + 3)
            pltpu.make_async_copy(src_hbm.at[pl.ds(o2, CH)], src_b[e0], ss[e0]).wait()
            pltpu.make_async_copy(dst_hbm.at[pl.ds(o2, CH)], dst_b[e0], sd[e0]).wait()
            pltpu.make_async_copy(src_hbm.at[pl.ds(o3, CH)], src_b[e1], ss[e1]).wait()
            pltpu.make_async_copy(dst_hbm.at[pl.ds(o3, CH)], dst_b[e1], sd[e1]).wait()
            pltpu.async_copy(y_sh.at[src_b[e0]], rows_b[e0], sg[e0])
            pltpu.async_copy(y_sh.at[src_b[e1]], rows_b[e1], sg[e1])
            pltpu.async_copy(rows_b[a], z_sh.at[dst_b[a]], sc0, add=True)
            pltpu.async_copy(rows_b[b], z_sh.at[dst_b[b]], sc1, add=True)
            pltpu.make_async_copy(rows_b[a], z_sh.at[dst_b[a]], sc0).wait()
            pltpu.make_async_copy(rows_b[b], z_sh.at[dst_b[b]], sc1).wait()
            o4, o5 = idx_off(2 * g + 4), idx_off(2 * g + 5)
            pltpu.async_copy(src_hbm.at[pl.ds(o4, CH)], src_b[a], ss[a])
            pltpu.async_copy(dst_hbm.at[pl.ds(o4, CH)], dst_b[a], sd[a])
            pltpu.async_copy(src_hbm.at[pl.ds(o5, CH)], src_b[b], ss[b])
            pltpu.async_copy(dst_hbm.at[pl.ds(o5, CH)], dst_b[b], sd[b])

        def ebody(gg, carry):
            step(2 * gg, 0)
            step(2 * gg + 1, 1)
            return carry

        lax.fori_loop(0, ng // 2, ebody, 0)
        p_last = (ng - 1) & 1
        q_last = 1 - p_last
        for i in (2 * q_last, 2 * q_last + 1):
            pltpu.make_async_copy(y_sh.at[src_b[i]], rows_b[i], sg[i]).wait()
        for i in (2 * p_last, 2 * p_last + 1):
            pltpu.make_async_copy(src_hbm.at[pl.ds(last, CH)], src_b[i], ss[i]).wait()
            pltpu.make_async_copy(dst_hbm.at[pl.ds(last, CH)], dst_b[i], sd[i]).wait()
        plsc.subcore_barrier()
        pltpu.sync_copy(z_sh.at[pl.ds(base, rps)],
                        out_hbm.at[pl.ds(c * np_ + base, rps)])

    return pl.kernel(
        body,
        out_type=jax.ShapeDtypeStruct((NCORE * np_, h), jnp.float32),
        mesh=mesh,
        compiler_params=pltpu.CompilerParams(use_tc_tiling_on_sc=False),
        scratch_types=(
            [pltpu.VMEM((CH,), jnp.int32)] * 8
            + [pltpu.VMEM((CH, h), jnp.float32)] * 4
            + [pltpu.VMEM((zr, h), jnp.float32),
               pltpu.VMEM_SHARED((np_, h), jnp.float32),
               pltpu.VMEM_SHARED((np_, h), jnp.float32)]
            + [pltpu.SemaphoreType.DMA] * 14
        ),
    )


@functools.lru_cache(None)
def _dinv_kernel(np_):
    def body(degp_ref, out_ref):
        deg = degp_ref[0, :] + degp_ref[1, :] + 1.0
        out_ref[...] = lax.rsqrt(deg)

    return pl.pallas_call(
        body, out_shape=jax.ShapeDtypeStruct((np_,), jnp.float32))


@functools.lru_cache(None)
def _first_tc(np_, f_in, h1):
    r = 1024

    def body(x_ref, w_ref, dinv_ref, out_ref):
        xw = jnp.dot(x_ref[...], w_ref[...],
                     preferred_element_type=jnp.float32,
                     precision=lax.Precision.HIGHEST)
        out_ref[...] = xw * dinv_ref[...]

    return pl.pallas_call(
        body,
        grid=(np_ // r,),
        in_specs=[
            pl.BlockSpec((r, f_in), lambda i: (i, 0)),
            pl.BlockSpec((f_in, h1), lambda i: (0, 0)),
            pl.BlockSpec((r, 1), lambda i: (i, 0)),
        ],
        out_specs=pl.BlockSpec((r, h1), lambda i: (i, 0)),
        out_shape=jax.ShapeDtypeStruct((np_, h1), jnp.float32),
    )


@functools.lru_cache(None)
def _mid_tc(np_, hin, hout):
    r = 1024

    def body(z0_ref, z1_ref, y_ref, b_ref, w_ref, dinv_ref, out_ref):
        dinv = dinv_ref[...]
        agg = z0_ref[...] + z1_ref[...] + y_ref[...]
        hcur = jnp.maximum(agg * dinv + b_ref[...], 0.0)
        out_ref[...] = jnp.dot(hcur, w_ref[...],
                               preferred_element_type=jnp.float32,
                               precision=lax.Precision.HIGHEST) * dinv

    return pl.pallas_call(
        body,
        grid=(np_ // r,),
        in_specs=[
            pl.BlockSpec((r, hin), lambda i: (i, 0)),
            pl.BlockSpec((r, hin), lambda i: (i, 0)),
            pl.BlockSpec((r, hin), lambda i: (i, 0)),
            pl.BlockSpec((1, hin), lambda i: (0, 0)),
            pl.BlockSpec((hin, hout), lambda i: (0, 0)),
            pl.BlockSpec((r, 1), lambda i: (i, 0)),
        ],
        out_specs=pl.BlockSpec((r, hout), lambda i: (i, 0)),
        out_shape=jax.ShapeDtypeStruct((np_, hout), jnp.float32),
    )


@functools.lru_cache(None)
def _final_tc(n, hp, c_out):
    r = 1000

    def body(z0_ref, z1_ref, y_ref, b_ref, dinv_ref, out_ref):
        agg = z0_ref[...] + z1_ref[...] + y_ref[...]
        res = agg * dinv_ref[...] + b_ref[...]
        out_ref[...] = res[:, :c_out]

    return pl.pallas_call(
        body,
        grid=(n // r,),
        in_specs=[
            pl.BlockSpec((r, hp), lambda i: (i, 0)),
            pl.BlockSpec((r, hp), lambda i: (i, 0)),
            pl.BlockSpec((r, hp), lambda i: (i, 0)),
            pl.BlockSpec((1, hp), lambda i: (0, 0)),
            pl.BlockSpec((r, 1), lambda i: (i, 0)),
        ],
        out_specs=pl.BlockSpec((r, c_out), lambda i: (i, 0)),
        out_shape=jax.ShapeDtypeStruct((n, c_out), jnp.float32),
    )


def kernel(x, edge_index, W1, b1, W2, b2, W3, b3):
    n, f_in = x.shape
    e = edge_index.shape[1]
    h1, h2, c_out = W1.shape[1], W2.shape[1], W3.shape[1]
    hp = -(-c_out // 16) * 16

    np_ = (n // 512 + 1) * 512
    ep = -(-e // (NW * CH * 4)) * (NW * CH * 4)

    src = edge_index[0]
    dst = edge_index[1]
    pad = ep - e
    if pad:
        fill = jnp.full((pad,), n, dtype=src.dtype)
        src = jnp.concatenate([src, fill])
        dst = jnp.concatenate([dst, fill])
    xp = jnp.pad(x, ((0, np_ - n), (0, 0)))
    w3p = jnp.pad(W3, ((0, 0), (0, hp - c_out)))
    b3p = jnp.pad(b3, (0, hp - c_out)).reshape(1, hp)

    degp = _deg_kernel(np_, ep)(dst)
    dinv2 = _dinv_kernel(np_)(degp.reshape(NCORE, np_)).reshape(np_, 1)

    y1 = _first_tc(np_, f_in, h1)(xp, W1, dinv2)
    z1 = _agg_kernel(np_, ep, h1)(y1, src, dst)
    y2 = _mid_tc(np_, h1, h2)(z1[:np_], z1[np_:], y1, b1.reshape(1, h1), W2, dinv2)
    z2 = _agg_kernel(np_, ep, h2)(y2, src, dst)
    y3 = _mid_tc(np_, h2, hp)(z2[:np_], z2[np_:], y2, b2.reshape(1, h2), w3p, dinv2)
    z3 = _agg_kernel(np_, ep, hp)(y3, src, dst)
    return _final_tc(n, hp, c_out)(z3[:np_], z3[np_:], y3, b3p, dinv2)
